# Initial kernel scaffold; baseline (speedup 1.0000x reference)
import jax
import jax.numpy as jnp
from jax.experimental import pallas as pl


def kernel(x, w1, w2):
    raise NotImplementedError("write your pallas kernel here")


# speedup vs baseline: 1.1089x; 1.1089x over previous
import functools

import jax
import jax.numpy as jnp
from jax.experimental import pallas as pl
from jax.experimental.pallas import tpu as pltpu

_VMEM_LIMIT_BYTES = 64 * 1024 * 1024


def _se_fused_kernel(x_ref, w1_ref, w2_ref, o_ref, *, inv_hw):
    xb = x_ref[...]
    pooled = jnp.sum(xb, axis=1, keepdims=True) * inv_hw
    h = jax.lax.dot_general(
        w1_ref[...], pooled, (((0,), (0,)), ((), ())),
        preferred_element_type=jnp.float32)
    h = jnp.maximum(h, 0.0)
    y = jax.lax.dot_general(
        w2_ref[...], h, (((0,), (0,)), ((), ())),
        preferred_element_type=jnp.float32)
    o_ref[...] = xb * jax.nn.sigmoid(y)


@jax.jit
def kernel(x, w1, w2):
    b, c, hh, ww = x.shape
    hw = hh * ww
    x2 = x.reshape(b * c, hw)

    out2 = pl.pallas_call(
        functools.partial(_se_fused_kernel, inv_hw=1.0 / hw),
        out_shape=jax.ShapeDtypeStruct((b * c, hw), x.dtype),
        grid=(b,),
        in_specs=[
            pl.BlockSpec((c, hw), lambda i: (i, 0)),
            pl.BlockSpec(w1.shape, lambda i: (0, 0)),
            pl.BlockSpec(w2.shape, lambda i: (0, 0)),
        ],
        out_specs=pl.BlockSpec((c, hw), lambda i: (i, 0)),
        compiler_params=pltpu.CompilerParams(
            dimension_semantics=("parallel",),
            vmem_limit_bytes=_VMEM_LIMIT_BYTES,
        ),
    )(x2, w1, w2)

    return out2.reshape(b, c, hh, ww)


# speedup vs baseline: 6.4769x; 5.8409x over previous
import functools

import jax
import jax.numpy as jnp
from jax.experimental import pallas as pl
from jax.experimental.pallas import tpu as pltpu

_VMEM_LIMIT_BYTES = 64 * 1024 * 1024


def _se_fused_kernel(x_ref, w1t_ref, w2_ref, o_ref, *, inv_hw):
    xb = x_ref[0]
    pooled = jnp.sum(xb, axis=0, keepdims=True) * inv_hw
    h = jax.lax.dot_general(
        pooled, w1t_ref[...], (((1,), (1,)), ((), ())),
        preferred_element_type=jnp.float32)
    h = jnp.maximum(h, 0.0)
    y = jnp.dot(h, w2_ref[...], preferred_element_type=jnp.float32)
    o_ref[0] = xb * jax.nn.sigmoid(y)


@jax.jit
def kernel(x, w1, w2):
    b, c, hh, ww = x.shape
    hw = hh * ww
    xt = jnp.transpose(x, (0, 2, 3, 1)).reshape(b, hw, c)
    w1t = jnp.transpose(w1)

    out = pl.pallas_call(
        functools.partial(_se_fused_kernel, inv_hw=1.0 / hw),
        out_shape=jax.ShapeDtypeStruct((b, hw, c), x.dtype),
        grid=(b,),
        in_specs=[
            pl.BlockSpec((1, hw, c), lambda i: (i, 0, 0)),
            pl.BlockSpec(w1t.shape, lambda i: (0, 0)),
            pl.BlockSpec(w2.shape, lambda i: (0, 0)),
        ],
        out_specs=pl.BlockSpec((1, hw, c), lambda i: (i, 0, 0)),
        compiler_params=pltpu.CompilerParams(
            dimension_semantics=("parallel",),
            vmem_limit_bytes=_VMEM_LIMIT_BYTES,
        ),
    )(xt, w1t, w2)

    return jnp.transpose(out.reshape(b, hh, ww, c), (0, 3, 1, 2))


# speedup vs baseline: 9.1164x; 1.4075x over previous
import functools

import jax
import jax.numpy as jnp
from jax.experimental import pallas as pl
from jax.experimental.pallas import tpu as pltpu

_VMEM_LIMIT_BYTES = 64 * 1024 * 1024


_BATCH_BLOCK = 2


def _se_fused_kernel(x_ref, w1t_ref, w2_ref, o_ref, *, inv_hw):
    xb = x_ref[...]
    pooled = jnp.sum(xb, axis=1) * inv_hw
    h = jax.lax.dot_general(
        pooled, w1t_ref[...], (((1,), (1,)), ((), ())),
        preferred_element_type=jnp.float32)
    h = jnp.maximum(h, 0.0)
    y = jnp.dot(h, w2_ref[...], preferred_element_type=jnp.float32)
    o_ref[...] = xb * jax.nn.sigmoid(y)[:, None, :]


@jax.jit
def kernel(x, w1, w2):
    b, c, hh, ww = x.shape
    hw = hh * ww
    xt = jnp.transpose(x, (0, 2, 3, 1)).reshape(b, hw, c)
    w1t = jnp.transpose(w1)

    out = pl.pallas_call(
        functools.partial(_se_fused_kernel, inv_hw=1.0 / hw),
        out_shape=jax.ShapeDtypeStruct((b, hw, c), x.dtype),
        grid=(b // _BATCH_BLOCK,),
        in_specs=[
            pl.BlockSpec((_BATCH_BLOCK, hw, c), lambda i: (i, 0, 0)),
            pl.BlockSpec(w1t.shape, lambda i: (0, 0)),
            pl.BlockSpec(w2.shape, lambda i: (0, 0)),
        ],
        out_specs=pl.BlockSpec((_BATCH_BLOCK, hw, c), lambda i: (i, 0, 0)),
        compiler_params=pltpu.CompilerParams(
            dimension_semantics=("parallel",),
            vmem_limit_bytes=_VMEM_LIMIT_BYTES,
        ),
    )(xt, w1t, w2)

    return jnp.transpose(out.reshape(b, hh, ww, c), (0, 3, 1, 2))


# speedup vs baseline: 10.7504x; 1.1792x over previous
import functools

import jax
import jax.numpy as jnp
from jax.experimental import pallas as pl
from jax.experimental.pallas import tpu as pltpu

_VMEM_LIMIT_BYTES = 64 * 1024 * 1024


_BATCH_BLOCK = 4


def _se_fused_kernel(x_ref, w1t_ref, w2_ref, o_ref, *, inv_hw):
    xb = x_ref[...]
    pooled = jnp.sum(xb, axis=1) * inv_hw
    h = jax.lax.dot_general(
        pooled, w1t_ref[...], (((1,), (1,)), ((), ())),
        preferred_element_type=jnp.float32)
    h = jnp.maximum(h, 0.0)
    y = jnp.dot(h, w2_ref[...], preferred_element_type=jnp.float32)
    o_ref[...] = xb * jax.nn.sigmoid(y)[:, None, :]


@jax.jit
def kernel(x, w1, w2):
    b, c, hh, ww = x.shape
    hw = hh * ww
    xt = jnp.transpose(x, (0, 2, 3, 1)).reshape(b, hw, c)
    w1t = jnp.transpose(w1)

    out = pl.pallas_call(
        functools.partial(_se_fused_kernel, inv_hw=1.0 / hw),
        out_shape=jax.ShapeDtypeStruct((b, hw, c), x.dtype),
        grid=(b // _BATCH_BLOCK,),
        in_specs=[
            pl.BlockSpec((_BATCH_BLOCK, hw, c), lambda i: (i, 0, 0)),
            pl.BlockSpec(w1t.shape, lambda i: (0, 0)),
            pl.BlockSpec(w2.shape, lambda i: (0, 0)),
        ],
        out_specs=pl.BlockSpec((_BATCH_BLOCK, hw, c), lambda i: (i, 0, 0)),
        compiler_params=pltpu.CompilerParams(
            dimension_semantics=("parallel",),
            vmem_limit_bytes=_VMEM_LIMIT_BYTES,
        ),
    )(xt, w1t, w2)

    return jnp.transpose(out.reshape(b, hh, ww, c), (0, 3, 1, 2))


# speedup vs baseline: 11.5130x; 1.0709x over previous
import functools

import jax
import jax.numpy as jnp
from jax.experimental import pallas as pl
from jax.experimental.pallas import tpu as pltpu

_VMEM_LIMIT_BYTES = 64 * 1024 * 1024


_BATCH_BLOCK = 8


def _se_fused_kernel(x_ref, w1t_ref, w2_ref, o_ref, *, inv_hw):
    xb = x_ref[...]
    pooled = jnp.sum(xb, axis=1) * inv_hw
    h = jax.lax.dot_general(
        pooled, w1t_ref[...], (((1,), (1,)), ((), ())),
        preferred_element_type=jnp.float32)
    h = jnp.maximum(h, 0.0)
    y = jnp.dot(h, w2_ref[...], preferred_element_type=jnp.float32)
    o_ref[...] = xb * jax.nn.sigmoid(y)[:, None, :]


@jax.jit
def kernel(x, w1, w2):
    b, c, hh, ww = x.shape
    hw = hh * ww
    xt = jnp.transpose(x, (0, 2, 3, 1)).reshape(b, hw, c)
    w1t = jnp.transpose(w1)

    out = pl.pallas_call(
        functools.partial(_se_fused_kernel, inv_hw=1.0 / hw),
        out_shape=jax.ShapeDtypeStruct((b, hw, c), x.dtype),
        grid=(b // _BATCH_BLOCK,),
        in_specs=[
            pl.BlockSpec((_BATCH_BLOCK, hw, c), lambda i: (i, 0, 0)),
            pl.BlockSpec(w1t.shape, lambda i: (0, 0)),
            pl.BlockSpec(w2.shape, lambda i: (0, 0)),
        ],
        out_specs=pl.BlockSpec((_BATCH_BLOCK, hw, c), lambda i: (i, 0, 0)),
        compiler_params=pltpu.CompilerParams(
            dimension_semantics=("parallel",),
            vmem_limit_bytes=_VMEM_LIMIT_BYTES,
        ),
    )(xt, w1t, w2)

    return jnp.transpose(out.reshape(b, hh, ww, c), (0, 3, 1, 2))


# speedup vs baseline: 11.5276x; 1.0013x over previous
import functools
import math

import jax
import jax.numpy as jnp
from jax.experimental import pallas as pl
from jax.experimental.pallas import tpu as pltpu

_VMEM_LIMIT_BYTES = 64 * 1024 * 1024
_BATCH_BLOCK = 8


def _se_fused_kernel(x_ref, w1t_ref, w2_ref, o_ref, *, inv_hw):
    xb = x_ref[...]
    pooled = jnp.sum(xb, axis=1) * inv_hw
    h = jax.lax.dot_general(
        pooled, w1t_ref[...], (((1,), (1,)), ((), ())),
        preferred_element_type=jnp.float32)
    h = jnp.maximum(h, 0.0)
    y = jnp.dot(h, w2_ref[...], preferred_element_type=jnp.float32)
    o_ref[...] = xb * jax.nn.sigmoid(y)[:, None, :]


@jax.jit
def kernel(x, w1, w2):
    b, c, hh, ww = x.shape
    hw = hh * ww
    nb = math.gcd(_BATCH_BLOCK, b)
    xt = jnp.transpose(x, (0, 2, 3, 1)).reshape(b, hw, c)
    w1t = jnp.transpose(w1)

    out = pl.pallas_call(
        functools.partial(_se_fused_kernel, inv_hw=1.0 / hw),
        out_shape=jax.ShapeDtypeStruct((b, hw, c), x.dtype),
        grid=(b // nb,),
        in_specs=[
            pl.BlockSpec((nb, hw, c), lambda i: (i, 0, 0)),
            pl.BlockSpec(w1t.shape, lambda i: (0, 0)),
            pl.BlockSpec(w2.shape, lambda i: (0, 0)),
        ],
        out_specs=pl.BlockSpec((nb, hw, c), lambda i: (i, 0, 0)),
        compiler_params=pltpu.CompilerParams(
            dimension_semantics=("parallel",),
            vmem_limit_bytes=_VMEM_LIMIT_BYTES,
        ),
    )(xt, w1t, w2)

    return jnp.transpose(out.reshape(b, hh, ww, c), (0, 3, 1, 2))
